# baseline (speedup 1.0000x reference)
import functools

import jax
import jax.numpy as jnp
from jax.experimental import pallas as pl
from jax.experimental.pallas import tpu as pltpu

E = 8
K = 2
D = 768
H = 2048
N = 2048
TN = 256


def _gate_weights(scores):
    ids = jax.lax.broadcasted_iota(jnp.int32, scores.shape, 1)
    m1 = jnp.max(scores, axis=-1, keepdims=True)
    i1 = jnp.min(jnp.where(scores == m1, ids, E), axis=-1, keepdims=True)
    s2 = jnp.where(ids == i1, -jnp.inf, scores)
    m2 = jnp.max(s2, axis=-1, keepdims=True)
    i2 = jnp.min(jnp.where(s2 == m2, ids, E), axis=-1, keepdims=True)
    mask = (ids == i1) | (ids == i2)
    p = jnp.exp(scores - m1)
    p = p / jnp.sum(p, axis=-1, keepdims=True)
    w = jnp.where(mask, p, 0.0)
    return w / (jnp.sum(w, axis=-1, keepdims=True) + 1e-8)


def _moe_body(x_ref, Wg_ref, bg_ref, W1_ref, b1_ref, W2_ref, b2_ref,
              out_ref, acc_ref):
    e = pl.program_id(0)
    t = pl.program_id(1)
    xt = x_ref[pl.ds(t * TN, TN), :]

    scores = jnp.dot(xt, Wg_ref[...], preferred_element_type=jnp.float32)
    scores = scores + bg_ref[0]
    w = _gate_weights(scores)
    eids = jax.lax.broadcasted_iota(jnp.int32, w.shape, 1)
    we = jnp.sum(jnp.where(eids == e, w, 0.0), axis=-1, keepdims=True)

    xb = xt.astype(jnp.bfloat16)
    h = jnp.dot(xb, W1_ref[0].astype(jnp.bfloat16),
                preferred_element_type=jnp.float32) + b1_ref[0, 0]
    h = jnp.maximum(h, 0.0)
    y = jnp.dot(h.astype(jnp.bfloat16), W2_ref[0].astype(jnp.bfloat16),
                preferred_element_type=jnp.float32) + b2_ref[0, 0]
    y = we * y

    @pl.when(e == 0)
    def _():
        acc_ref[pl.ds(t * TN, TN), :] = y

    @pl.when(e > 0)
    def _():
        acc_ref[pl.ds(t * TN, TN), :] += y

    @pl.when(e == E - 1)
    def _():
        out_ref[...] = acc_ref[pl.ds(t * TN, TN), :]


@jax.jit
def kernel(x, W_gate, b_gate, W1, b1, W2, b2):
    grid = (E, N // TN)
    return pl.pallas_call(
        _moe_body,
        grid=grid,
        in_specs=[
            pl.BlockSpec((N, D), lambda e, t: (0, 0)),
            pl.BlockSpec((D, E), lambda e, t: (0, 0)),
            pl.BlockSpec((1, E), lambda e, t: (0, 0)),
            pl.BlockSpec((1, D, H), lambda e, t: (e, 0, 0)),
            pl.BlockSpec((1, 1, H), lambda e, t: (e, 0, 0)),
            pl.BlockSpec((1, H, D), lambda e, t: (e, 0, 0)),
            pl.BlockSpec((1, 1, D), lambda e, t: (e, 0, 0)),
        ],
        out_specs=pl.BlockSpec((TN, D), lambda e, t: (t, 0)),
        out_shape=jax.ShapeDtypeStruct((N, D), jnp.float32),
        scratch_shapes=[pltpu.VMEM((N, D), jnp.float32)],
        compiler_params=pltpu.CompilerParams(
            dimension_semantics=("arbitrary", "arbitrary"),
        ),
    )(x, W_gate, b_gate.reshape(1, E), W1, b1.reshape(E, 1, H), W2,
      b2.reshape(E, 1, D))


# speedup vs baseline: 1.3074x; 1.3074x over previous
import functools

import jax
import jax.numpy as jnp
from jax import lax
from jax.experimental import pallas as pl
from jax.experimental.pallas import tpu as pltpu
from jax.experimental.pallas import tpu_sc as plsc

E = 8
K = 2
D = 768
H = 2048
N = 2048

TB = 256
NB = (N * K) // TB + E
P = NB * TB

NW = 32
TPW = N // NW


def _router_body(x_ref, Wg_ref, bg_ref, pos_ref, w_ref, be_ref, act_ref):
    x = x_ref[...]
    scores = jnp.dot(x, Wg_ref[...], preferred_element_type=jnp.float32)
    scores = scores + bg_ref[0]

    ids = lax.broadcasted_iota(jnp.int32, scores.shape, 1)
    m1 = jnp.max(scores, axis=-1, keepdims=True)
    i1 = jnp.min(jnp.where(scores == m1, ids, E), axis=-1, keepdims=True)
    s2 = jnp.where(ids == i1, -jnp.inf, scores)
    m2 = jnp.max(s2, axis=-1, keepdims=True)
    i2 = jnp.min(jnp.where(s2 == m2, ids, E), axis=-1, keepdims=True)
    sel1 = ids == i1
    sel2 = ids == i2
    p = jnp.exp(scores - m1)
    p = p / jnp.sum(p, axis=-1, keepdims=True)
    w = jnp.where(sel1 | sel2, p, 0.0)
    w = w / (jnp.sum(w, axis=-1, keepdims=True) + 1e-8)

    a = (sel1 | sel2).astype(jnp.float32)
    r = lax.broadcasted_iota(jnp.int32, (N, N), 0)
    c = lax.broadcasted_iota(jnp.int32, (N, N), 1)
    ltri = (c < r).astype(jnp.float32)
    ranks = jnp.dot(ltri, a, preferred_element_type=jnp.float32,
                    precision=lax.Precision.HIGHEST)
    counts = jnp.sum(a, axis=0, keepdims=True)
    pc = jnp.ceil(counts / TB) * TB
    eu = lax.broadcasted_iota(jnp.int32, (E, E), 0)
    ev = lax.broadcasted_iota(jnp.int32, (E, E), 1)
    triu = (eu < ev).astype(jnp.float32)
    off = jnp.dot(pc, triu, preferred_element_type=jnp.float32,
                  precision=lax.Precision.HIGHEST)

    slot = off + ranks
    pos0 = jnp.sum(jnp.where(sel1, slot, 0.0), axis=1)
    pos1 = jnp.sum(jnp.where(sel2, slot, 0.0), axis=1)
    w0 = jnp.sum(jnp.where(sel1, w, 0.0), axis=1)
    w1 = jnp.sum(jnp.where(sel2, w, 0.0), axis=1)
    pos_ref[...] = jnp.concatenate(
        [pos0[None].astype(jnp.int32), pos1[None].astype(jnp.int32)], axis=0)
    w_ref[...] = jnp.concatenate(
        [jnp.broadcast_to(w0[None, :, None], (1, N, 16)),
         jnp.broadcast_to(w1[None, :, None], (1, N, 16))], axis=0)

    ends = (off + pc).reshape(E, 1)
    bstart = (lax.broadcasted_iota(jnp.int32, (1, NB), 1) * TB).astype(
        jnp.float32)
    be = jnp.sum((ends <= bstart).astype(jnp.int32), axis=0, keepdims=True)
    total = jnp.sum(pc, axis=1, keepdims=True)
    act_ref[...] = (bstart < total).astype(jnp.int32)
    be_ref[...] = jnp.minimum(be, E - 1)


def _run_router(x, W_gate, b_gate):
    return pl.pallas_call(
        _router_body,
        in_specs=[
            pl.BlockSpec((N, D), lambda: (0, 0)),
            pl.BlockSpec((D, E), lambda: (0, 0)),
            pl.BlockSpec((1, E), lambda: (0, 0)),
        ],
        out_specs=[
            pl.BlockSpec((K, N), lambda: (0, 0)),
            pl.BlockSpec((K, N, 16), lambda: (0, 0, 0)),
            pl.BlockSpec((1, NB), lambda: (0, 0)),
            pl.BlockSpec((1, NB), lambda: (0, 0)),
        ],
        out_shape=[
            jax.ShapeDtypeStruct((K, N), jnp.int32),
            jax.ShapeDtypeStruct((K, N, 16), jnp.float32),
            jax.ShapeDtypeStruct((1, NB), jnp.int32),
            jax.ShapeDtypeStruct((1, NB), jnp.int32),
        ],
    )(x, W_gate, b_gate.reshape(1, E))


def _make_scatter():
    mesh = plsc.VectorSubcoreMesh(core_axis_name="c", subcore_axis_name="s")

    @functools.partial(
        pl.kernel, mesh=mesh,
        out_type=jax.ShapeDtypeStruct((P, D), jnp.float32),
        scratch_types=[
            pltpu.VMEM((TPW,), jnp.int32),
            pltpu.VMEM((TPW,), jnp.int32),
            pltpu.VMEM((TPW, D), jnp.float32),
            pltpu.SemaphoreType.DMA,
        ],
    )
    def scatter(x_hbm, pos_hbm, xs_hbm, idx0_v, idx1_v, buf_v, sem):
        wid = lax.axis_index("s") * 2 + lax.axis_index("c")
        base = wid * TPW
        pltpu.sync_copy(pos_hbm.at[0, pl.ds(base, TPW)], idx0_v)
        pltpu.sync_copy(pos_hbm.at[1, pl.ds(base, TPW)], idx1_v)
        pltpu.sync_copy(x_hbm.at[pl.ds(base, TPW)], buf_v)
        pltpu.async_copy(buf_v, xs_hbm.at[idx0_v], sem).wait()
        pltpu.async_copy(buf_v, xs_hbm.at[idx1_v], sem).wait()

    return scatter


def _ffn_body(be_ref, act_ref, xs_ref, W1_ref, b1_ref, W2_ref, b2_ref,
              ys_ref):
    i = pl.program_id(0)

    @pl.when(act_ref[i] == 1)
    def _():
        h = jnp.dot(xs_ref[...], W1_ref[0],
                    preferred_element_type=jnp.float32) + b1_ref[0, 0]
        h = jnp.maximum(h, 0.0)
        ys_ref[...] = jnp.dot(h, W2_ref[0],
                              preferred_element_type=jnp.float32) + b2_ref[0, 0]


def _run_ffn(be, act, xs, W1, b1, W2, b2):
    grid_spec = pltpu.PrefetchScalarGridSpec(
        num_scalar_prefetch=2,
        grid=(NB,),
        in_specs=[
            pl.BlockSpec((TB, D), lambda i, be, act: (i, 0)),
            pl.BlockSpec((1, D, H), lambda i, be, act: (be[i], 0, 0)),
            pl.BlockSpec((1, 1, H), lambda i, be, act: (be[i], 0, 0)),
            pl.BlockSpec((1, H, D), lambda i, be, act: (be[i], 0, 0)),
            pl.BlockSpec((1, 1, D), lambda i, be, act: (be[i], 0, 0)),
        ],
        out_specs=pl.BlockSpec((TB, D), lambda i, be, act: (i, 0)),
    )
    return pl.pallas_call(
        _ffn_body,
        grid_spec=grid_spec,
        out_shape=jax.ShapeDtypeStruct((P, D), jnp.float32),
        compiler_params=pltpu.CompilerParams(
            dimension_semantics=("arbitrary",),
        ),
    )(be, act, xs, W1, b1.reshape(E, 1, H), W2, b2.reshape(E, 1, D))


def _make_combine():
    mesh = plsc.VectorSubcoreMesh(core_axis_name="c", subcore_axis_name="s")

    @functools.partial(
        pl.kernel, mesh=mesh,
        out_type=jax.ShapeDtypeStruct((N, D), jnp.float32),
        scratch_types=[
            pltpu.VMEM((TPW,), jnp.int32),
            pltpu.VMEM((TPW,), jnp.int32),
            pltpu.VMEM((TPW, 16), jnp.float32),
            pltpu.VMEM((TPW, 16), jnp.float32),
            pltpu.VMEM((TPW, D), jnp.float32),
            pltpu.VMEM((TPW, D), jnp.float32),
            pltpu.SemaphoreType.DMA,
        ],
    )
    def combine(ys_hbm, pos_hbm, w_hbm, out_hbm,
                idx0_v, idx1_v, w0_v, w1_v, bufa, bufb, sem):
        wid = lax.axis_index("s") * 2 + lax.axis_index("c")
        base = wid * TPW
        pltpu.sync_copy(pos_hbm.at[0, pl.ds(base, TPW)], idx0_v)
        pltpu.sync_copy(pos_hbm.at[1, pl.ds(base, TPW)], idx1_v)
        pltpu.sync_copy(w_hbm.at[0, pl.ds(base, TPW)], w0_v)
        pltpu.sync_copy(w_hbm.at[1, pl.ds(base, TPW)], w1_v)
        pltpu.async_copy(ys_hbm.at[idx0_v], bufa, sem).wait()
        pltpu.async_copy(ys_hbm.at[idx1_v], bufb, sem).wait()

        def body(i, carry):
            s0 = w0_v[i, :]
            s1 = w1_v[i, :]
            for j in range(D // 16):
                av = bufa[i, pl.ds(j * 16, 16)]
                bv = bufb[i, pl.ds(j * 16, 16)]
                bufa[i, pl.ds(j * 16, 16)] = s0 * av + s1 * bv
            return carry

        lax.fori_loop(0, TPW, body, 0)
        pltpu.sync_copy(bufa, out_hbm.at[pl.ds(base, TPW)])

    return combine


_make_scatter = functools.cache(_make_scatter)
_make_combine = functools.cache(_make_combine)


@jax.jit
def kernel(x, W_gate, b_gate, W1, b1, W2, b2):
    pos, w, be, act = _run_router(x, W_gate, b_gate)
    xs = _make_scatter()(x, pos)
    ys = _run_ffn(be.reshape(NB), act.reshape(NB), xs, W1, b1, W2, b2)
    return _make_combine()(ys, pos, w)


# speedup vs baseline: 1.4290x; 1.0930x over previous
import functools

import jax
import jax.numpy as jnp
from jax import lax
from jax.experimental import pallas as pl
from jax.experimental.pallas import tpu as pltpu
from jax.experimental.pallas import tpu_sc as plsc

E = 8
K = 2
D = 768
H = 2048
N = 2048

TB = 256
NB = (N * K) // TB + E
P = NB * TB

NW = 32
TPW = N // NW


def _router_body(x_ref, Wg_ref, bg_ref, pos_ref, w_ref, be_ref, act_ref):
    x = x_ref[...]
    scores = jnp.dot(x, Wg_ref[...], preferred_element_type=jnp.float32)
    scores = scores + bg_ref[0]

    ids = lax.broadcasted_iota(jnp.int32, scores.shape, 1)
    m1 = jnp.max(scores, axis=-1, keepdims=True)
    i1 = jnp.min(jnp.where(scores == m1, ids, E), axis=-1, keepdims=True)
    s2 = jnp.where(ids == i1, -jnp.inf, scores)
    m2 = jnp.max(s2, axis=-1, keepdims=True)
    i2 = jnp.min(jnp.where(s2 == m2, ids, E), axis=-1, keepdims=True)
    sel1 = ids == i1
    sel2 = ids == i2
    p = jnp.exp(scores - m1)
    p = p / jnp.sum(p, axis=-1, keepdims=True)
    w = jnp.where(sel1 | sel2, p, 0.0)
    w = w / (jnp.sum(w, axis=-1, keepdims=True) + 1e-8)

    a = (sel1 | sel2).astype(jnp.float32)
    r = lax.broadcasted_iota(jnp.int32, (N, N), 0)
    c = lax.broadcasted_iota(jnp.int32, (N, N), 1)
    ltri = (c < r).astype(jnp.float32)
    ranks = jnp.dot(ltri, a, preferred_element_type=jnp.float32)
    counts = jnp.sum(a, axis=0, keepdims=True)
    pc = jnp.ceil(counts / TB) * TB
    eu = lax.broadcasted_iota(jnp.int32, (E, E), 0)
    ev = lax.broadcasted_iota(jnp.int32, (E, E), 1)
    triu = (eu < ev).astype(jnp.float32)
    off = jnp.dot(pc, triu, preferred_element_type=jnp.float32)

    slot = off + ranks
    pos0 = jnp.sum(jnp.where(sel1, slot, 0.0), axis=1)
    pos1 = jnp.sum(jnp.where(sel2, slot, 0.0), axis=1)
    w0 = jnp.sum(jnp.where(sel1, w, 0.0), axis=1)
    w1 = jnp.sum(jnp.where(sel2, w, 0.0), axis=1)
    pos_ref[...] = jnp.concatenate(
        [pos0[None].astype(jnp.int32), pos1[None].astype(jnp.int32)], axis=0)
    w_ref[...] = jnp.concatenate(
        [jnp.broadcast_to(w0[None, :, None], (1, N, 16)),
         jnp.broadcast_to(w1[None, :, None], (1, N, 16))], axis=0)

    ends = (off + pc).reshape(E, 1)
    bstart = (lax.broadcasted_iota(jnp.int32, (1, NB), 1) * TB).astype(
        jnp.float32)
    be = jnp.sum((ends <= bstart).astype(jnp.int32), axis=0, keepdims=True)
    total = jnp.sum(pc, axis=1, keepdims=True)
    act_ref[...] = (bstart < total).astype(jnp.int32)
    be_ref[...] = jnp.minimum(be, E - 1)


def _run_router(x, W_gate, b_gate):
    return pl.pallas_call(
        _router_body,
        in_specs=[
            pl.BlockSpec((N, D), lambda: (0, 0)),
            pl.BlockSpec((D, E), lambda: (0, 0)),
            pl.BlockSpec((1, E), lambda: (0, 0)),
        ],
        out_specs=[
            pl.BlockSpec((K, N), lambda: (0, 0)),
            pl.BlockSpec((K, N, 16), lambda: (0, 0, 0)),
            pl.BlockSpec((1, NB), lambda: (0, 0)),
            pl.BlockSpec((1, NB), lambda: (0, 0)),
        ],
        out_shape=[
            jax.ShapeDtypeStruct((K, N), jnp.int32),
            jax.ShapeDtypeStruct((K, N, 16), jnp.float32),
            jax.ShapeDtypeStruct((1, NB), jnp.int32),
            jax.ShapeDtypeStruct((1, NB), jnp.int32),
        ],
    )(x, W_gate, b_gate.reshape(1, E))


def _make_scatter():
    mesh = plsc.VectorSubcoreMesh(core_axis_name="c", subcore_axis_name="s")

    @functools.partial(
        pl.kernel, mesh=mesh,
        out_type=jax.ShapeDtypeStruct((P, D), jnp.float32),
        scratch_types=[
            pltpu.VMEM((TPW,), jnp.int32),
            pltpu.VMEM((TPW,), jnp.int32),
            pltpu.VMEM((TPW, D), jnp.float32),
            pltpu.SemaphoreType.DMA,
        ],
    )
    def scatter(x_hbm, pos_hbm, xs_hbm, idx0_v, idx1_v, buf_v, sem):
        wid = lax.axis_index("s") * 2 + lax.axis_index("c")
        base = wid * TPW
        pltpu.sync_copy(pos_hbm.at[0, pl.ds(base, TPW)], idx0_v)
        pltpu.sync_copy(pos_hbm.at[1, pl.ds(base, TPW)], idx1_v)
        pltpu.sync_copy(x_hbm.at[pl.ds(base, TPW)], buf_v)
        c0 = pltpu.async_copy(buf_v, xs_hbm.at[idx0_v], sem)
        c1 = pltpu.async_copy(buf_v, xs_hbm.at[idx1_v], sem)
        c0.wait()
        c1.wait()

    return scatter


def _ffn_body(be_ref, act_ref, xs_ref, W1_ref, b1_ref, W2_ref, b2_ref,
              ys_ref):
    i = pl.program_id(0)

    @pl.when(act_ref[0, i] == 1)
    def _():
        h = jnp.dot(xs_ref[...], W1_ref[0],
                    preferred_element_type=jnp.float32) + b1_ref[0, 0]
        h = jnp.maximum(h, 0.0)
        ys_ref[...] = jnp.dot(h, W2_ref[0],
                              preferred_element_type=jnp.float32) + b2_ref[0, 0]


def _run_ffn(be, act, xs, W1, b1, W2, b2):
    grid_spec = pltpu.PrefetchScalarGridSpec(
        num_scalar_prefetch=2,
        grid=(NB,),
        in_specs=[
            pl.BlockSpec((TB, D), lambda i, be, act: (i, 0)),
            pl.BlockSpec((1, D, H), lambda i, be, act: (be[0, i], 0, 0)),
            pl.BlockSpec((1, 1, H), lambda i, be, act: (be[0, i], 0, 0)),
            pl.BlockSpec((1, H, D), lambda i, be, act: (be[0, i], 0, 0)),
            pl.BlockSpec((1, 1, D), lambda i, be, act: (be[0, i], 0, 0)),
        ],
        out_specs=pl.BlockSpec((TB, D), lambda i, be, act: (i, 0)),
    )
    return pl.pallas_call(
        _ffn_body,
        grid_spec=grid_spec,
        out_shape=jax.ShapeDtypeStruct((P, D), jnp.float32),
        compiler_params=pltpu.CompilerParams(
            dimension_semantics=("arbitrary",),
        ),
    )(be, act, xs, W1, b1.reshape(E, 1, H), W2, b2.reshape(E, 1, D))


def _make_combine():
    mesh = plsc.VectorSubcoreMesh(core_axis_name="c", subcore_axis_name="s")

    @functools.partial(
        pl.kernel, mesh=mesh,
        out_type=jax.ShapeDtypeStruct((N, D), jnp.float32),
        scratch_types=[
            pltpu.VMEM((TPW,), jnp.int32),
            pltpu.VMEM((TPW,), jnp.int32),
            pltpu.VMEM((TPW, 16), jnp.float32),
            pltpu.VMEM((TPW, 16), jnp.float32),
            pltpu.VMEM((TPW, D), jnp.float32),
            pltpu.VMEM((TPW, D), jnp.float32),
            pltpu.SemaphoreType.DMA,
        ],
    )
    def combine(ys_hbm, pos_hbm, w_hbm, out_hbm,
                idx0_v, idx1_v, w0_v, w1_v, bufa, bufb, sem):
        wid = lax.axis_index("s") * 2 + lax.axis_index("c")
        base = wid * TPW
        pltpu.sync_copy(pos_hbm.at[0, pl.ds(base, TPW)], idx0_v)
        pltpu.sync_copy(pos_hbm.at[1, pl.ds(base, TPW)], idx1_v)
        pltpu.sync_copy(w_hbm.at[0, pl.ds(base, TPW)], w0_v)
        pltpu.sync_copy(w_hbm.at[1, pl.ds(base, TPW)], w1_v)
        c0 = pltpu.async_copy(ys_hbm.at[idx0_v], bufa, sem)
        c1 = pltpu.async_copy(ys_hbm.at[idx1_v], bufb, sem)
        c0.wait()
        c1.wait()

        def body(i, carry):
            s0 = w0_v[i, :]
            s1 = w1_v[i, :]
            for j in range(D // 16):
                av = bufa[i, pl.ds(j * 16, 16)]
                bv = bufb[i, pl.ds(j * 16, 16)]
                bufa[i, pl.ds(j * 16, 16)] = s0 * av + s1 * bv
            return carry

        lax.fori_loop(0, TPW, body, 0)
        pltpu.sync_copy(bufa, out_hbm.at[pl.ds(base, TPW)])

    return combine


_make_scatter = functools.cache(_make_scatter)
_make_combine = functools.cache(_make_combine)


@jax.jit
def kernel(x, W_gate, b_gate, W1, b1, W2, b2):
    pos, w, be, act = _run_router(x, W_gate, b_gate)
    xs = _make_scatter()(x, pos)
    ys = _run_ffn(be, act, xs, W1, b1, W2, b2)
    return _make_combine()(ys, pos, w)


# speedup vs baseline: 1.4352x; 1.0043x over previous
import functools

import jax
import jax.numpy as jnp
from jax import lax
from jax.experimental import pallas as pl
from jax.experimental.pallas import tpu as pltpu
from jax.experimental.pallas import tpu_sc as plsc

E = 8
K = 2
D = 768
H = 2048
N = 2048

TB = 256
NB = (N * K) // TB + E
P = NB * TB

NW = 32
TPW = N // NW


def _router_body(x_ref, Wg_ref, bg_ref, pos_ref, w_ref, be_ref, act_ref):
    x = x_ref[...]
    scores = jnp.dot(x, Wg_ref[...], preferred_element_type=jnp.float32)
    scores = scores + bg_ref[0]

    ids = lax.broadcasted_iota(jnp.int32, scores.shape, 1)
    m1 = jnp.max(scores, axis=-1, keepdims=True)
    i1 = jnp.min(jnp.where(scores == m1, ids, E), axis=-1, keepdims=True)
    s2 = jnp.where(ids == i1, -jnp.inf, scores)
    m2 = jnp.max(s2, axis=-1, keepdims=True)
    i2 = jnp.min(jnp.where(s2 == m2, ids, E), axis=-1, keepdims=True)
    sel1 = ids == i1
    sel2 = ids == i2
    p = jnp.exp(scores - m1)
    p = p / jnp.sum(p, axis=-1, keepdims=True)
    w = jnp.where(sel1 | sel2, p, 0.0)
    w = w / (jnp.sum(w, axis=-1, keepdims=True) + 1e-8)

    CH = 256
    NCH = N // CH
    a = (sel1 | sel2).astype(jnp.float32)
    r = lax.broadcasted_iota(jnp.int32, (CH, CH), 0)
    c = lax.broadcasted_iota(jnp.int32, (CH, CH), 1)
    ltri = (c < r).astype(jnp.float32)
    chunk_ranks = [
        jnp.dot(ltri, a[k * CH:(k + 1) * CH], preferred_element_type=jnp.float32)
        for k in range(NCH)
    ]
    tot = jnp.concatenate(
        [jnp.sum(a[k * CH:(k + 1) * CH], axis=0, keepdims=True)
         for k in range(NCH)], axis=0)
    cu = lax.broadcasted_iota(jnp.int32, (NCH, NCH), 0)
    cv = lax.broadcasted_iota(jnp.int32, (NCH, NCH), 1)
    ctri = (cv < cu).astype(jnp.float32)
    chunk_off = jnp.dot(ctri, tot, preferred_element_type=jnp.float32)
    ranks = jnp.concatenate(
        [chunk_ranks[k] + chunk_off[k:k + 1] for k in range(NCH)], axis=0)
    counts = jnp.sum(tot, axis=0, keepdims=True)
    pc = jnp.ceil(counts / TB) * TB
    eu = lax.broadcasted_iota(jnp.int32, (E, E), 0)
    ev = lax.broadcasted_iota(jnp.int32, (E, E), 1)
    triu = (eu < ev).astype(jnp.float32)
    off = jnp.dot(pc, triu, preferred_element_type=jnp.float32)

    slot = off + ranks
    pos0 = jnp.sum(jnp.where(sel1, slot, 0.0), axis=1)
    pos1 = jnp.sum(jnp.where(sel2, slot, 0.0), axis=1)
    w0 = jnp.sum(jnp.where(sel1, w, 0.0), axis=1)
    w1 = jnp.sum(jnp.where(sel2, w, 0.0), axis=1)
    pos_ref[...] = jnp.concatenate(
        [pos0[None].astype(jnp.int32), pos1[None].astype(jnp.int32)], axis=0)
    w_ref[...] = jnp.concatenate(
        [jnp.broadcast_to(w0[None, :, None], (1, N, 16)),
         jnp.broadcast_to(w1[None, :, None], (1, N, 16))], axis=0)

    ends = (off + pc).reshape(E, 1)
    bstart = (lax.broadcasted_iota(jnp.int32, (1, NB), 1) * TB).astype(
        jnp.float32)
    be = jnp.sum((ends <= bstart).astype(jnp.int32), axis=0, keepdims=True)
    total = jnp.sum(pc, axis=1, keepdims=True)
    act_ref[...] = (bstart < total).astype(jnp.int32)
    be_ref[...] = jnp.minimum(be, E - 1)


def _run_router(x, W_gate, b_gate):
    return pl.pallas_call(
        _router_body,
        in_specs=[
            pl.BlockSpec((N, D), lambda: (0, 0)),
            pl.BlockSpec((D, E), lambda: (0, 0)),
            pl.BlockSpec((1, E), lambda: (0, 0)),
        ],
        out_specs=[
            pl.BlockSpec((K, N), lambda: (0, 0)),
            pl.BlockSpec((K, N, 16), lambda: (0, 0, 0)),
            pl.BlockSpec((1, NB), lambda: (0, 0)),
            pl.BlockSpec((1, NB), lambda: (0, 0)),
        ],
        out_shape=[
            jax.ShapeDtypeStruct((K, N), jnp.int32),
            jax.ShapeDtypeStruct((K, N, 16), jnp.float32),
            jax.ShapeDtypeStruct((1, NB), jnp.int32),
            jax.ShapeDtypeStruct((1, NB), jnp.int32),
        ],
    )(x, W_gate, b_gate.reshape(1, E))


def _make_scatter():
    mesh = plsc.VectorSubcoreMesh(core_axis_name="c", subcore_axis_name="s")

    @functools.partial(
        pl.kernel, mesh=mesh,
        out_type=jax.ShapeDtypeStruct((P, D), jnp.float32),
        scratch_types=[
            pltpu.VMEM((TPW,), jnp.int32),
            pltpu.VMEM((TPW,), jnp.int32),
            pltpu.VMEM((TPW, D), jnp.float32),
            pltpu.SemaphoreType.DMA,
        ],
    )
    def scatter(x_hbm, pos_hbm, xs_hbm, idx0_v, idx1_v, buf_v, sem):
        wid = lax.axis_index("s") * 2 + lax.axis_index("c")
        base = wid * TPW
        pltpu.sync_copy(pos_hbm.at[0, pl.ds(base, TPW)], idx0_v)
        pltpu.sync_copy(pos_hbm.at[1, pl.ds(base, TPW)], idx1_v)
        pltpu.sync_copy(x_hbm.at[pl.ds(base, TPW)], buf_v)
        c0 = pltpu.async_copy(buf_v, xs_hbm.at[idx0_v], sem)
        c1 = pltpu.async_copy(buf_v, xs_hbm.at[idx1_v], sem)
        c0.wait()
        c1.wait()

    return scatter


def _ffn_body(be_ref, act_ref, xs_ref, W1_ref, b1_ref, W2_ref, b2_ref,
              ys_ref):
    i = pl.program_id(0)

    @pl.when(act_ref[0, i] == 1)
    def _():
        h = jnp.dot(xs_ref[...], W1_ref[0],
                    preferred_element_type=jnp.float32) + b1_ref[0, 0]
        h = jnp.maximum(h, 0.0)
        ys_ref[...] = jnp.dot(h, W2_ref[0],
                              preferred_element_type=jnp.float32) + b2_ref[0, 0]


def _run_ffn(be, act, xs, W1, b1, W2, b2):
    grid_spec = pltpu.PrefetchScalarGridSpec(
        num_scalar_prefetch=2,
        grid=(NB,),
        in_specs=[
            pl.BlockSpec((TB, D), lambda i, be, act: (i, 0)),
            pl.BlockSpec((1, D, H), lambda i, be, act: (be[0, i], 0, 0)),
            pl.BlockSpec((1, 1, H), lambda i, be, act: (be[0, i], 0, 0)),
            pl.BlockSpec((1, H, D), lambda i, be, act: (be[0, i], 0, 0)),
            pl.BlockSpec((1, 1, D), lambda i, be, act: (be[0, i], 0, 0)),
        ],
        out_specs=pl.BlockSpec((TB, D), lambda i, be, act: (i, 0)),
    )
    return pl.pallas_call(
        _ffn_body,
        grid_spec=grid_spec,
        out_shape=jax.ShapeDtypeStruct((P, D), jnp.float32),
        compiler_params=pltpu.CompilerParams(
            dimension_semantics=("arbitrary",),
        ),
    )(be, act, xs, W1, b1.reshape(E, 1, H), W2, b2.reshape(E, 1, D))


def _make_combine():
    mesh = plsc.VectorSubcoreMesh(core_axis_name="c", subcore_axis_name="s")

    @functools.partial(
        pl.kernel, mesh=mesh,
        out_type=jax.ShapeDtypeStruct((N, D), jnp.float32),
        scratch_types=[
            pltpu.VMEM((TPW,), jnp.int32),
            pltpu.VMEM((TPW,), jnp.int32),
            pltpu.VMEM((TPW, 16), jnp.float32),
            pltpu.VMEM((TPW, 16), jnp.float32),
            pltpu.VMEM((TPW, D), jnp.float32),
            pltpu.VMEM((TPW, D), jnp.float32),
            pltpu.SemaphoreType.DMA,
        ],
    )
    def combine(ys_hbm, pos_hbm, w_hbm, out_hbm,
                idx0_v, idx1_v, w0_v, w1_v, bufa, bufb, sem):
        wid = lax.axis_index("s") * 2 + lax.axis_index("c")
        base = wid * TPW
        pltpu.sync_copy(pos_hbm.at[0, pl.ds(base, TPW)], idx0_v)
        pltpu.sync_copy(pos_hbm.at[1, pl.ds(base, TPW)], idx1_v)
        pltpu.sync_copy(w_hbm.at[0, pl.ds(base, TPW)], w0_v)
        pltpu.sync_copy(w_hbm.at[1, pl.ds(base, TPW)], w1_v)
        c0 = pltpu.async_copy(ys_hbm.at[idx0_v], bufa, sem)
        c1 = pltpu.async_copy(ys_hbm.at[idx1_v], bufb, sem)
        c0.wait()
        c1.wait()

        def body(i, carry):
            s0 = w0_v[i, :]
            s1 = w1_v[i, :]
            for j in range(D // 16):
                av = bufa[i, pl.ds(j * 16, 16)]
                bv = bufb[i, pl.ds(j * 16, 16)]
                bufa[i, pl.ds(j * 16, 16)] = s0 * av + s1 * bv
            return carry

        lax.fori_loop(0, TPW, body, 0)
        pltpu.sync_copy(bufa, out_hbm.at[pl.ds(base, TPW)])

    return combine


_make_scatter = functools.cache(_make_scatter)
_make_combine = functools.cache(_make_combine)


@jax.jit
def kernel(x, W_gate, b_gate, W1, b1, W2, b2):
    pos, w, be, act = _run_router(x, W_gate, b_gate)
    xs = _make_scatter()(x, pos)
    ys = _run_ffn(be, act, xs, W1, b1, W2, b2)
    return _make_combine()(ys, pos, w)


# speedup vs baseline: 1.5541x; 1.0829x over previous
import functools

import jax
import jax.numpy as jnp
from jax import lax
from jax.experimental import pallas as pl
from jax.experimental.pallas import tpu as pltpu
from jax.experimental.pallas import tpu_sc as plsc

E = 8
K = 2
D = 768
H = 2048
N = 2048

TB = 512
NB = (N * K) // TB + E
P = NB * TB

NW = 32
TPW = N // NW


def _router_body(x_ref, Wg_ref, bg_ref, pos_ref, w_ref, be_ref, act_ref):
    x = x_ref[...]
    scores = jnp.dot(x, Wg_ref[...], preferred_element_type=jnp.float32)
    scores = scores + bg_ref[0]

    ids = lax.broadcasted_iota(jnp.int32, scores.shape, 1)
    m1 = jnp.max(scores, axis=-1, keepdims=True)
    i1 = jnp.min(jnp.where(scores == m1, ids, E), axis=-1, keepdims=True)
    s2 = jnp.where(ids == i1, -jnp.inf, scores)
    m2 = jnp.max(s2, axis=-1, keepdims=True)
    i2 = jnp.min(jnp.where(s2 == m2, ids, E), axis=-1, keepdims=True)
    sel1 = ids == i1
    sel2 = ids == i2
    p = jnp.exp(scores - m1)
    p = p / jnp.sum(p, axis=-1, keepdims=True)
    w = jnp.where(sel1 | sel2, p, 0.0)
    w = w / (jnp.sum(w, axis=-1, keepdims=True) + 1e-8)

    CH = 256
    NCH = N // CH
    a = (sel1 | sel2).astype(jnp.float32)
    r = lax.broadcasted_iota(jnp.int32, (CH, CH), 0)
    c = lax.broadcasted_iota(jnp.int32, (CH, CH), 1)
    ltri = (c < r).astype(jnp.float32)
    chunk_ranks = [
        jnp.dot(ltri, a[k * CH:(k + 1) * CH], preferred_element_type=jnp.float32)
        for k in range(NCH)
    ]
    tot = jnp.concatenate(
        [jnp.sum(a[k * CH:(k + 1) * CH], axis=0, keepdims=True)
         for k in range(NCH)], axis=0)
    cu = lax.broadcasted_iota(jnp.int32, (NCH, NCH), 0)
    cv = lax.broadcasted_iota(jnp.int32, (NCH, NCH), 1)
    ctri = (cv < cu).astype(jnp.float32)
    chunk_off = jnp.dot(ctri, tot, preferred_element_type=jnp.float32)
    ranks = jnp.concatenate(
        [chunk_ranks[k] + chunk_off[k:k + 1] for k in range(NCH)], axis=0)
    counts = jnp.sum(tot, axis=0, keepdims=True)
    pc = jnp.ceil(counts / TB) * TB
    eu = lax.broadcasted_iota(jnp.int32, (E, E), 0)
    ev = lax.broadcasted_iota(jnp.int32, (E, E), 1)
    triu = (eu < ev).astype(jnp.float32)
    off = jnp.dot(pc, triu, preferred_element_type=jnp.float32)

    slot = off + ranks
    pos0 = jnp.sum(jnp.where(sel1, slot, 0.0), axis=1)
    pos1 = jnp.sum(jnp.where(sel2, slot, 0.0), axis=1)
    w0 = jnp.sum(jnp.where(sel1, w, 0.0), axis=1)
    w1 = jnp.sum(jnp.where(sel2, w, 0.0), axis=1)
    pos_ref[...] = jnp.concatenate(
        [pos0[None].astype(jnp.int32), pos1[None].astype(jnp.int32)], axis=0)
    w_ref[...] = jnp.concatenate(
        [jnp.broadcast_to(w0[None, :, None], (1, N, 16)),
         jnp.broadcast_to(w1[None, :, None], (1, N, 16))], axis=0)

    ends = (off + pc).reshape(E, 1)
    bstart = (lax.broadcasted_iota(jnp.int32, (1, NB), 1) * TB).astype(
        jnp.float32)
    be = jnp.sum((ends <= bstart).astype(jnp.int32), axis=0, keepdims=True)
    total = jnp.sum(pc, axis=1, keepdims=True)
    act_ref[...] = (bstart < total).astype(jnp.int32)
    be_ref[...] = jnp.minimum(be, E - 1)


def _run_router(x, W_gate, b_gate):
    return pl.pallas_call(
        _router_body,
        in_specs=[
            pl.BlockSpec((N, D), lambda: (0, 0)),
            pl.BlockSpec((D, E), lambda: (0, 0)),
            pl.BlockSpec((1, E), lambda: (0, 0)),
        ],
        out_specs=[
            pl.BlockSpec((K, N), lambda: (0, 0)),
            pl.BlockSpec((K, N, 16), lambda: (0, 0, 0)),
            pl.BlockSpec((1, NB), lambda: (0, 0)),
            pl.BlockSpec((1, NB), lambda: (0, 0)),
        ],
        out_shape=[
            jax.ShapeDtypeStruct((K, N), jnp.int32),
            jax.ShapeDtypeStruct((K, N, 16), jnp.float32),
            jax.ShapeDtypeStruct((1, NB), jnp.int32),
            jax.ShapeDtypeStruct((1, NB), jnp.int32),
        ],
    )(x, W_gate, b_gate.reshape(1, E))


def _make_scatter():
    mesh = plsc.VectorSubcoreMesh(core_axis_name="c", subcore_axis_name="s")

    @functools.partial(
        pl.kernel, mesh=mesh,
        out_type=jax.ShapeDtypeStruct((P, D), jnp.float32),
        scratch_types=[
            pltpu.VMEM((TPW,), jnp.int32),
            pltpu.VMEM((TPW,), jnp.int32),
            pltpu.VMEM((TPW, D), jnp.float32),
            pltpu.SemaphoreType.DMA,
        ],
    )
    def scatter(x_hbm, pos_hbm, xs_hbm, idx0_v, idx1_v, buf_v, sem):
        wid = lax.axis_index("s") * 2 + lax.axis_index("c")
        base = wid * TPW
        pltpu.sync_copy(pos_hbm.at[0, pl.ds(base, TPW)], idx0_v)
        pltpu.sync_copy(pos_hbm.at[1, pl.ds(base, TPW)], idx1_v)
        pltpu.sync_copy(x_hbm.at[pl.ds(base, TPW)], buf_v)
        c0 = pltpu.async_copy(buf_v, xs_hbm.at[idx0_v], sem)
        c1 = pltpu.async_copy(buf_v, xs_hbm.at[idx1_v], sem)
        c0.wait()
        c1.wait()

    return scatter


def _ffn_body(be_ref, act_ref, xs_ref, W1_ref, b1_ref, W2_ref, b2_ref,
              ys_ref):
    i = pl.program_id(0)

    @pl.when(act_ref[0, i] == 1)
    def _():
        h = jnp.dot(xs_ref[...], W1_ref[0],
                    preferred_element_type=jnp.float32) + b1_ref[0, 0]
        h = jnp.maximum(h, 0.0)
        ys_ref[...] = jnp.dot(h, W2_ref[0],
                              preferred_element_type=jnp.float32) + b2_ref[0, 0]


def _run_ffn(be, act, xs, W1, b1, W2, b2):
    grid_spec = pltpu.PrefetchScalarGridSpec(
        num_scalar_prefetch=2,
        grid=(NB,),
        in_specs=[
            pl.BlockSpec((TB, D), lambda i, be, act: (i, 0)),
            pl.BlockSpec((1, D, H), lambda i, be, act: (be[0, i], 0, 0)),
            pl.BlockSpec((1, 1, H), lambda i, be, act: (be[0, i], 0, 0)),
            pl.BlockSpec((1, H, D), lambda i, be, act: (be[0, i], 0, 0)),
            pl.BlockSpec((1, 1, D), lambda i, be, act: (be[0, i], 0, 0)),
        ],
        out_specs=pl.BlockSpec((TB, D), lambda i, be, act: (i, 0)),
    )
    return pl.pallas_call(
        _ffn_body,
        grid_spec=grid_spec,
        out_shape=jax.ShapeDtypeStruct((P, D), jnp.float32),
        compiler_params=pltpu.CompilerParams(
            dimension_semantics=("arbitrary",),
        ),
    )(be, act, xs, W1, b1.reshape(E, 1, H), W2, b2.reshape(E, 1, D))


def _make_combine():
    mesh = plsc.VectorSubcoreMesh(core_axis_name="c", subcore_axis_name="s")

    @functools.partial(
        pl.kernel, mesh=mesh,
        out_type=jax.ShapeDtypeStruct((N, D), jnp.float32),
        scratch_types=[
            pltpu.VMEM((TPW,), jnp.int32),
            pltpu.VMEM((TPW,), jnp.int32),
            pltpu.VMEM((TPW, 16), jnp.float32),
            pltpu.VMEM((TPW, 16), jnp.float32),
            pltpu.VMEM((TPW, D), jnp.float32),
            pltpu.VMEM((TPW, D), jnp.float32),
            pltpu.SemaphoreType.DMA,
        ],
    )
    def combine(ys_hbm, pos_hbm, w_hbm, out_hbm,
                idx0_v, idx1_v, w0_v, w1_v, bufa, bufb, sem):
        wid = lax.axis_index("s") * 2 + lax.axis_index("c")
        base = wid * TPW
        pltpu.sync_copy(pos_hbm.at[0, pl.ds(base, TPW)], idx0_v)
        pltpu.sync_copy(pos_hbm.at[1, pl.ds(base, TPW)], idx1_v)
        pltpu.sync_copy(w_hbm.at[0, pl.ds(base, TPW)], w0_v)
        pltpu.sync_copy(w_hbm.at[1, pl.ds(base, TPW)], w1_v)
        c0 = pltpu.async_copy(ys_hbm.at[idx0_v], bufa, sem)
        c1 = pltpu.async_copy(ys_hbm.at[idx1_v], bufb, sem)
        c0.wait()
        c1.wait()

        def body(i, carry):
            s0 = w0_v[i, :]
            s1 = w1_v[i, :]
            for j in range(D // 16):
                av = bufa[i, pl.ds(j * 16, 16)]
                bv = bufb[i, pl.ds(j * 16, 16)]
                bufa[i, pl.ds(j * 16, 16)] = s0 * av + s1 * bv
            return carry

        lax.fori_loop(0, TPW, body, 0)
        pltpu.sync_copy(bufa, out_hbm.at[pl.ds(base, TPW)])

    return combine


_make_scatter = functools.cache(_make_scatter)
_make_combine = functools.cache(_make_combine)


@jax.jit
def kernel(x, W_gate, b_gate, W1, b1, W2, b2):
    pos, w, be, act = _run_router(x, W_gate, b_gate)
    xs = _make_scatter()(x, pos)
    ys = _run_ffn(be, act, xs, W1, b1, W2, b2)
    return _make_combine()(ys, pos, w)


# speedup vs baseline: 1.6014x; 1.0304x over previous
import functools

import jax
import jax.numpy as jnp
from jax import lax
from jax.experimental import pallas as pl
from jax.experimental.pallas import tpu as pltpu
from jax.experimental.pallas import tpu_sc as plsc

E = 8
K = 2
D = 768
H = 2048
N = 2048

TB = 512
NB = (N * K) // TB + E
P = NB * TB

NW = 32
TPW = N // NW


def _router_body(x_ref, Wg_ref, bg_ref, pos_ref, w_ref, be_ref, act_ref):
    x = x_ref[...]
    scores = jnp.dot(x, Wg_ref[...], preferred_element_type=jnp.float32)
    scores = scores + bg_ref[0]

    ids = lax.broadcasted_iota(jnp.int32, scores.shape, 1)
    m1 = jnp.max(scores, axis=-1, keepdims=True)
    i1 = jnp.min(jnp.where(scores == m1, ids, E), axis=-1, keepdims=True)
    s2 = jnp.where(ids == i1, -jnp.inf, scores)
    m2 = jnp.max(s2, axis=-1, keepdims=True)
    i2 = jnp.min(jnp.where(s2 == m2, ids, E), axis=-1, keepdims=True)
    sel1 = ids == i1
    sel2 = ids == i2
    p = jnp.exp(scores - m1)
    p = p / jnp.sum(p, axis=-1, keepdims=True)
    w = jnp.where(sel1 | sel2, p, 0.0)
    w = w / (jnp.sum(w, axis=-1, keepdims=True) + 1e-8)

    CH = 256
    NCH = N // CH
    a = (sel1 | sel2).astype(jnp.float32)
    r = lax.broadcasted_iota(jnp.int32, (CH, CH), 0)
    c = lax.broadcasted_iota(jnp.int32, (CH, CH), 1)
    ltri = (c < r).astype(jnp.float32)
    chunk_ranks = [
        jnp.dot(ltri, a[k * CH:(k + 1) * CH], preferred_element_type=jnp.float32)
        for k in range(NCH)
    ]
    tot = jnp.concatenate(
        [jnp.sum(a[k * CH:(k + 1) * CH], axis=0, keepdims=True)
         for k in range(NCH)], axis=0)
    cu = lax.broadcasted_iota(jnp.int32, (NCH, NCH), 0)
    cv = lax.broadcasted_iota(jnp.int32, (NCH, NCH), 1)
    ctri = (cv < cu).astype(jnp.float32)
    chunk_off = jnp.dot(ctri, tot, preferred_element_type=jnp.float32)
    ranks = jnp.concatenate(
        [chunk_ranks[k] + chunk_off[k:k + 1] for k in range(NCH)], axis=0)
    counts = jnp.sum(tot, axis=0, keepdims=True)
    pc = jnp.ceil(counts / TB) * TB
    eu = lax.broadcasted_iota(jnp.int32, (E, E), 0)
    ev = lax.broadcasted_iota(jnp.int32, (E, E), 1)
    triu = (eu < ev).astype(jnp.float32)
    off = jnp.dot(pc, triu, preferred_element_type=jnp.float32)

    slot = off + ranks
    pos0 = jnp.sum(jnp.where(sel1, slot, 0.0), axis=1)
    pos1 = jnp.sum(jnp.where(sel2, slot, 0.0), axis=1)
    w0 = jnp.sum(jnp.where(sel1, w, 0.0), axis=1)
    w1 = jnp.sum(jnp.where(sel2, w, 0.0), axis=1)
    pos_ref[...] = jnp.concatenate(
        [pos0[None].astype(jnp.int32), pos1[None].astype(jnp.int32)], axis=0)
    w_ref[...] = jnp.concatenate(
        [jnp.broadcast_to(w0[None, :, None], (1, N, 16)),
         jnp.broadcast_to(w1[None, :, None], (1, N, 16))], axis=0)

    ends = (off + pc).reshape(E, 1)
    bstart = (lax.broadcasted_iota(jnp.int32, (1, NB), 1) * TB).astype(
        jnp.float32)
    be = jnp.sum((ends <= bstart).astype(jnp.int32), axis=0, keepdims=True)
    total = jnp.sum(pc, axis=1, keepdims=True)
    act_ref[...] = (bstart < total).astype(jnp.int32)
    be_ref[...] = jnp.minimum(be, E - 1)


def _run_router(x, W_gate, b_gate):
    return pl.pallas_call(
        _router_body,
        in_specs=[
            pl.BlockSpec((N, D), lambda: (0, 0)),
            pl.BlockSpec((D, E), lambda: (0, 0)),
            pl.BlockSpec((1, E), lambda: (0, 0)),
        ],
        out_specs=[
            pl.BlockSpec((K, N), lambda: (0, 0)),
            pl.BlockSpec((K, N, 16), lambda: (0, 0, 0)),
            pl.BlockSpec((1, NB), lambda: (0, 0)),
            pl.BlockSpec((1, NB), lambda: (0, 0)),
        ],
        out_shape=[
            jax.ShapeDtypeStruct((K, N), jnp.int32),
            jax.ShapeDtypeStruct((K, N, 16), jnp.float32),
            jax.ShapeDtypeStruct((1, NB), jnp.int32),
            jax.ShapeDtypeStruct((1, NB), jnp.int32),
        ],
    )(x, W_gate, b_gate.reshape(1, E))


def _make_scatter():
    mesh = plsc.VectorSubcoreMesh(core_axis_name="c", subcore_axis_name="s")

    @functools.partial(
        pl.kernel, mesh=mesh,
        out_type=jax.ShapeDtypeStruct((P, D), jnp.float32),
        scratch_types=[
            pltpu.VMEM((TPW,), jnp.int32),
            pltpu.VMEM((TPW,), jnp.int32),
            pltpu.VMEM((TPW, D), jnp.float32),
            pltpu.SemaphoreType.DMA,
        ],
    )
    def scatter(x_hbm, pos_hbm, xs_hbm, idx0_v, idx1_v, buf_v, sem):
        wid = lax.axis_index("s") * 2 + lax.axis_index("c")
        base = wid * TPW
        pltpu.sync_copy(pos_hbm.at[0, pl.ds(base, TPW)], idx0_v)
        pltpu.sync_copy(pos_hbm.at[1, pl.ds(base, TPW)], idx1_v)
        pltpu.sync_copy(x_hbm.at[pl.ds(base, TPW)], buf_v)
        c0 = pltpu.async_copy(buf_v, xs_hbm.at[idx0_v], sem)
        c1 = pltpu.async_copy(buf_v, xs_hbm.at[idx1_v], sem)
        c0.wait()
        c1.wait()

    return scatter


def _ffn_body(be_ref, act_ref, xs_ref, W1_ref, b1_ref, W2_ref, b2_ref,
              ys_ref):
    i = pl.program_id(0)

    @pl.when(act_ref[0, i] == 1)
    def _():
        h = jnp.dot(xs_ref[...], W1_ref[0],
                    preferred_element_type=jnp.float32) + b1_ref[0, 0]
        h = jnp.maximum(h, 0.0)
        ys_ref[...] = jnp.dot(h, W2_ref[0],
                              preferred_element_type=jnp.float32) + b2_ref[0, 0]


def _run_ffn(be, act, xs, W1, b1, W2, b2):
    grid_spec = pltpu.PrefetchScalarGridSpec(
        num_scalar_prefetch=2,
        grid=(NB,),
        in_specs=[
            pl.BlockSpec((TB, D),
                         lambda i, be, act: (jnp.where(act[0, i] == 1, i, 0), 0)),
            pl.BlockSpec((1, D, H), lambda i, be, act: (be[0, i], 0, 0)),
            pl.BlockSpec((1, 1, H), lambda i, be, act: (be[0, i], 0, 0)),
            pl.BlockSpec((1, H, D), lambda i, be, act: (be[0, i], 0, 0)),
            pl.BlockSpec((1, 1, D), lambda i, be, act: (be[0, i], 0, 0)),
        ],
        out_specs=pl.BlockSpec((TB, D), lambda i, be, act: (i, 0)),
    )
    return pl.pallas_call(
        _ffn_body,
        grid_spec=grid_spec,
        out_shape=jax.ShapeDtypeStruct((P, D), jnp.float32),
        compiler_params=pltpu.CompilerParams(
            dimension_semantics=("arbitrary",),
        ),
    )(be, act, xs, W1, b1.reshape(E, 1, H), W2, b2.reshape(E, 1, D))


def _make_combine():
    mesh = plsc.VectorSubcoreMesh(core_axis_name="c", subcore_axis_name="s")

    @functools.partial(
        pl.kernel, mesh=mesh,
        out_type=jax.ShapeDtypeStruct((N, D), jnp.float32),
        scratch_types=[
            pltpu.VMEM((TPW,), jnp.int32),
            pltpu.VMEM((TPW,), jnp.int32),
            pltpu.VMEM((TPW, 16), jnp.float32),
            pltpu.VMEM((TPW, 16), jnp.float32),
            pltpu.VMEM((TPW, D), jnp.float32),
            pltpu.VMEM((TPW, D), jnp.float32),
            pltpu.SemaphoreType.DMA,
        ],
    )
    def combine(ys_hbm, pos_hbm, w_hbm, out_hbm,
                idx0_v, idx1_v, w0_v, w1_v, bufa, bufb, sem):
        wid = lax.axis_index("s") * 2 + lax.axis_index("c")
        base = wid * TPW
        pltpu.sync_copy(pos_hbm.at[0, pl.ds(base, TPW)], idx0_v)
        pltpu.sync_copy(pos_hbm.at[1, pl.ds(base, TPW)], idx1_v)
        pltpu.sync_copy(w_hbm.at[0, pl.ds(base, TPW)], w0_v)
        pltpu.sync_copy(w_hbm.at[1, pl.ds(base, TPW)], w1_v)
        HF = TPW // 2
        cps = []
        for g in range(2):
            sl = pl.ds(g * HF, HF)
            cps.append(pltpu.async_copy(
                ys_hbm.at[idx0_v.at[sl]], bufa.at[sl], sem))
            cps.append(pltpu.async_copy(
                ys_hbm.at[idx1_v.at[sl]], bufb.at[sl], sem))

        def body(i, carry):
            s0 = w0_v[i, :]
            s1 = w1_v[i, :]
            for j in range(D // 16):
                av = bufa[i, pl.ds(j * 16, 16)]
                bv = bufb[i, pl.ds(j * 16, 16)]
                bufa[i, pl.ds(j * 16, 16)] = s0 * av + s1 * bv
            return carry

        cps[0].wait()
        cps[1].wait()
        lax.fori_loop(0, HF, body, 0)
        cps[2].wait()
        cps[3].wait()
        lax.fori_loop(HF, TPW, body, 0)
        pltpu.sync_copy(bufa, out_hbm.at[pl.ds(base, TPW)])

    return combine


_make_scatter = functools.cache(_make_scatter)
_make_combine = functools.cache(_make_combine)


@jax.jit
def kernel(x, W_gate, b_gate, W1, b1, W2, b2):
    pos, w, be, act = _run_router(x, W_gate, b_gate)
    xs = _make_scatter()(x, pos)
    ys = _run_ffn(be, act, xs, W1, b1, W2, b2)
    return _make_combine()(ys, pos, w)
